# baseline (speedup 1.0000x reference)
import functools

import jax
import jax.numpy as jnp
from jax import lax
from jax.experimental import pallas as pl
from jax.experimental.pallas import tpu as pltpu
from jax.experimental.pallas import tpu_sc as plsc

B = 8
H = W = 64
P = H * W
C3 = 3
NW = 32
PPW = (B * P) // NW
CHUNK = 16
NCH = PPW // CHUNK
NG = 4
CPG = NCH // NG

_OX = (0, 0, 1, 1)
_OY = (0, 1, 0, 1)


def _body(corr_hbm, aux_hbm, out_hbm,
          sc_v, gx_v, gy_v, vis_v, cidx_v, kw_v, g_v, out_v,
          sem0, sem1, sem2, sem3, semin, semout):
    cid = lax.axis_index("c")
    sid = lax.axis_index("s")
    wid = sid * 2 + cid
    b = wid // 4
    qw = wid % 4
    pq = qw * PPW
    abase = b * (6 * P)
    in_gx = pltpu.async_copy(aux_hbm.at[pl.ds(abase + pq, PPW)], gx_v, semin)
    in_gy = pltpu.async_copy(aux_hbm.at[pl.ds(abase + P + pq, PPW)], gy_v,
                             semin)
    in_vis = pltpu.async_copy(aux_hbm.at[pl.ds(abase + 2 * P + pq, PPW)],
                              vis_v, semin)
    in_sc = pltpu.async_copy(aux_hbm.at[pl.ds(abase + 3 * P, C3 * P)], sc_v,
                             semin)

    iota = lax.iota(jnp.int32, CHUNK)
    pconst = ((iota >> 3) << 15) + ((iota & 7) << 7)
    cbase = b * (P * P)

    def floorv(g):
        t = g.astype(jnp.int32)
        tf = t.astype(jnp.float32)
        return jnp.where(tf > g, t - 1, t)

    def phase_a(k):
        pb = k * CHUNK
        gx = (gx_v[pl.ds(pb, CHUNK)] + 1.0) * ((W - 1) * 0.5)
        gy = (gy_v[pl.ds(pb, CHUNK)] + 1.0) * ((H - 1) * 0.5)
        fxi = floorv(gx)
        fyi = floorv(gy)
        fxf = fxi.astype(jnp.float32)
        fyf = fyi.astype(jnp.float32)
        wy0 = fyf + 1.0 - gy
        wy1 = gy - fyf
        wx0 = fxf + 1.0 - gx
        wx1 = gx - fxf
        wts = (wy0 * wx0, wy0 * wx1, wy1 * wx0, wy1 * wx1)
        pscal = cbase + (((pq + pb) >> 3) << 15)
        pvec = pscal + pconst
        idxs = []
        for i in range(4):
            ix = jnp.clip(fyi + _OX[i], 0, H - 1)
            iy = jnp.clip(fxi + _OY[i], 0, W - 1)
            idxs.append(ix * W + iy)
        base = pb * 4
        for i in range(4):
            dup = idxs[i] != idxs[i]
            for j in range(i + 1, 4):
                dup = jnp.logical_or(dup, idxs[i] == idxs[j])
            keep = jnp.where(dup, 0.0, 1.0)
            off = base + i * CHUNK
            q = idxs[i]
            cidx_v[pl.ds(off, CHUNK)] = pvec + (((q >> 7) << 10) + (q & 127))
            kw_v[pl.ds(off, CHUNK)] = keep * wts[i]

    def phase_c(k):
        pb = k * CHUNK
        base = pb * 4
        vis = vis_v[pl.ds(pb, CHUNK)]
        cis = [cidx_v[pl.ds(base + i * CHUNK, CHUNK)] for i in range(4)]
        acc_s = [jnp.zeros((CHUNK,), jnp.float32) for _ in range(C3)]
        acc_g = [jnp.zeros((CHUNK,), jnp.float32) for _ in range(C3)]
        for i in range(4):
            off = base + i * CHUNK
            si = (((cis[i] >> 10) & 31) << 7) + (cis[i] & 127)
            dup = cis[i] != cis[i]
            for j in range(i + 1, 4):
                dup = jnp.logical_or(dup, cis[i] == cis[j])
            kw = kw_v[pl.ds(off, CHUNK)]
            kg = jnp.where(dup, 0.0, g_v[pl.ds(off, CHUNK)])
            for c in range(C3):
                s = plsc.load_gather(sc_v, [si + c * P])
                acc_s[c] = acc_s[c] + kw * s
                acc_g[c] = acc_g[c] + kg * s
        for c in range(C3):
            out_v[pl.ds(c * PPW + pb, CHUNK)] = acc_s[c] * vis
            out_v[pl.ds((C3 + c) * PPW + pb, CHUNK)] = acc_g[c] * vis

    sems = (sem0, sem1, sem2, sem3)
    copies = []
    gwords = 4 * CPG * CHUNK
    in_gx.wait()
    in_gy.wait()
    for g in range(NG):
        plsc.parallel_loop(g * CPG, (g + 1) * CPG, unroll=1)(phase_a)
        copies.append(pltpu.async_copy(
            corr_hbm.at[cidx_v.at[pl.ds(g * gwords, gwords)]],
            g_v.at[pl.ds(g * gwords, gwords)], sems[g]))
    in_vis.wait()
    in_sc.wait()
    outs = []
    for g in range(NG):
        copies[g].wait()
        plsc.parallel_loop(g * CPG, (g + 1) * CPG, unroll=1)(phase_c)
    for c in range(6):
        outs.append(pltpu.async_copy(
            out_v.at[pl.ds(c * PPW, PPW)],
            out_hbm.at[pl.ds((b * 6 + c) * P + pq, PPW)], semout))
    for o in outs:
        o.wait()


@jax.jit
def kernel(corr_m, gt_flow, vis_mask, scale_ref):
    mesh = plsc.VectorSubcoreMesh(core_axis_name="c", subcore_axis_name="s")
    run = functools.partial(
        pl.kernel,
        mesh=mesh,
        compiler_params=pltpu.CompilerParams(needs_layout_passes=False),
        out_type=jax.ShapeDtypeStruct((B * 6 * P,), jnp.float32),
        scratch_types=[
            pltpu.VMEM((C3 * P,), jnp.float32),
            pltpu.VMEM((PPW,), jnp.float32),
            pltpu.VMEM((PPW,), jnp.float32),
            pltpu.VMEM((PPW,), jnp.float32),
            pltpu.VMEM((4 * PPW,), jnp.int32),
            pltpu.VMEM((4 * PPW,), jnp.float32),
            pltpu.VMEM((4 * PPW,), jnp.float32),
            pltpu.VMEM((6 * PPW,), jnp.float32),
            pltpu.SemaphoreType.DMA,
            pltpu.SemaphoreType.DMA,
            pltpu.SemaphoreType.DMA,
            pltpu.SemaphoreType.DMA,
            pltpu.SemaphoreType.DMA,
            pltpu.SemaphoreType.DMA,
        ],
    )(_body)
    corr_tile_flat = jnp.transpose(
        corr_m.reshape(B, P // 8, 8, P // 128, 128), (0, 1, 3, 2, 4)
    ).reshape(B * P * P)
    aux = jnp.concatenate([gt_flow, vis_mask, scale_ref],
                          axis=1).reshape(B * 6 * P)
    out = run(corr_tile_flat, aux)
    return out.reshape(B, 6, H, W)


# speedup vs baseline: 1.0077x; 1.0077x over previous
import functools

import jax
import jax.numpy as jnp
from jax import lax
from jax.experimental import pallas as pl
from jax.experimental.pallas import tpu as pltpu
from jax.experimental.pallas import tpu_sc as plsc

B = 8
H = W = 64
P = H * W
C3 = 3
NW = 32
PPW = (B * P) // NW
CHUNK = 16
NCH = PPW // CHUNK
NG = 2
CPG = NCH // NG

_OX = (0, 0, 1, 1)
_OY = (0, 1, 0, 1)


def _body(corr_hbm, aux_hbm, out_hbm,
          sc_v, gx_v, gy_v, vis_v, cidx_v, kw_v, g_v, out_v,
          sem0, sem1, semin, semout):
    cid = lax.axis_index("c")
    sid = lax.axis_index("s")
    wid = sid * 2 + cid
    b = wid // 4
    qw = wid % 4
    pq = qw * PPW
    abase = b * (6 * P)
    in_gx = pltpu.async_copy(aux_hbm.at[pl.ds(abase + pq, PPW)], gx_v, semin)
    in_gy = pltpu.async_copy(aux_hbm.at[pl.ds(abase + P + pq, PPW)], gy_v,
                             semin)
    in_vis = pltpu.async_copy(aux_hbm.at[pl.ds(abase + 2 * P + pq, PPW)],
                              vis_v, semin)
    in_sc = pltpu.async_copy(aux_hbm.at[pl.ds(abase + 3 * P, C3 * P)], sc_v,
                             semin)

    iota = lax.iota(jnp.int32, CHUNK)
    pconst = ((iota >> 3) << 15) + ((iota & 7) << 7)
    cbase = b * (P * P)

    def floorv(g):
        t = g.astype(jnp.int32)
        tf = t.astype(jnp.float32)
        return jnp.where(tf > g, t - 1, t)

    def phase_a(k):
        pb = k * CHUNK
        gx = (gx_v[pl.ds(pb, CHUNK)] + 1.0) * ((W - 1) * 0.5)
        gy = (gy_v[pl.ds(pb, CHUNK)] + 1.0) * ((H - 1) * 0.5)
        fxi = floorv(gx)
        fyi = floorv(gy)
        fxf = fxi.astype(jnp.float32)
        fyf = fyi.astype(jnp.float32)
        wy0 = fyf + 1.0 - gy
        wy1 = gy - fyf
        wx0 = fxf + 1.0 - gx
        wx1 = gx - fxf
        wts = (wy0 * wx0, wy0 * wx1, wy1 * wx0, wy1 * wx1)
        pscal = cbase + (((pq + pb) >> 3) << 15)
        pvec = pscal + pconst
        idxs = []
        for i in range(4):
            ix = jnp.clip(fyi + _OX[i], 0, H - 1)
            iy = jnp.clip(fxi + _OY[i], 0, W - 1)
            idxs.append(ix * W + iy)
        base = pb * 4
        for i in range(4):
            dup = idxs[i] != idxs[i]
            for j in range(i + 1, 4):
                dup = jnp.logical_or(dup, idxs[i] == idxs[j])
            keep = jnp.where(dup, 0.0, 1.0)
            off = base + i * CHUNK
            q = idxs[i]
            cidx_v[pl.ds(off, CHUNK)] = pvec + (((q >> 7) << 10) + (q & 127))
            kw_v[pl.ds(off, CHUNK)] = keep * wts[i]

    def phase_c(k):
        pb = k * CHUNK
        base = pb * 4
        vis = vis_v[pl.ds(pb, CHUNK)]
        cis = [cidx_v[pl.ds(base + i * CHUNK, CHUNK)] for i in range(4)]
        acc_s = [jnp.zeros((CHUNK,), jnp.float32) for _ in range(C3)]
        acc_g = [jnp.zeros((CHUNK,), jnp.float32) for _ in range(C3)]
        for i in range(4):
            off = base + i * CHUNK
            si = (((cis[i] >> 10) & 31) << 7) + (cis[i] & 127)
            dup = cis[i] != cis[i]
            for j in range(i + 1, 4):
                dup = jnp.logical_or(dup, cis[i] == cis[j])
            kw = kw_v[pl.ds(off, CHUNK)]
            kg = jnp.where(dup, 0.0, g_v[pl.ds(off, CHUNK)])
            for c in range(C3):
                s = plsc.load_gather(sc_v, [si + c * P])
                acc_s[c] = acc_s[c] + kw * s
                acc_g[c] = acc_g[c] + kg * s
        for c in range(C3):
            out_v[pl.ds(c * PPW + pb, CHUNK)] = acc_s[c] * vis
            out_v[pl.ds((C3 + c) * PPW + pb, CHUNK)] = acc_g[c] * vis

    sems = (sem0, sem1)
    copies = []
    gwords = 4 * CPG * CHUNK
    in_gx.wait()
    in_gy.wait()
    for g in range(NG):
        plsc.parallel_loop(g * CPG, (g + 1) * CPG, unroll=1)(phase_a)
        copies.append(pltpu.async_copy(
            corr_hbm.at[cidx_v.at[pl.ds(g * gwords, gwords)]],
            g_v.at[pl.ds(g * gwords, gwords)], sems[g]))
    in_vis.wait()
    in_sc.wait()
    outs = []
    for g in range(NG):
        copies[g].wait()
        plsc.parallel_loop(g * CPG, (g + 1) * CPG, unroll=1)(phase_c)
    for c in range(6):
        outs.append(pltpu.async_copy(
            out_v.at[pl.ds(c * PPW, PPW)],
            out_hbm.at[pl.ds((b * 6 + c) * P + pq, PPW)], semout))
    for o in outs:
        o.wait()


@jax.jit
def kernel(corr_m, gt_flow, vis_mask, scale_ref):
    mesh = plsc.VectorSubcoreMesh(core_axis_name="c", subcore_axis_name="s")
    run = functools.partial(
        pl.kernel,
        mesh=mesh,
        compiler_params=pltpu.CompilerParams(needs_layout_passes=False),
        out_type=jax.ShapeDtypeStruct((B * 6 * P,), jnp.float32),
        scratch_types=[
            pltpu.VMEM((C3 * P,), jnp.float32),
            pltpu.VMEM((PPW,), jnp.float32),
            pltpu.VMEM((PPW,), jnp.float32),
            pltpu.VMEM((PPW,), jnp.float32),
            pltpu.VMEM((4 * PPW,), jnp.int32),
            pltpu.VMEM((4 * PPW,), jnp.float32),
            pltpu.VMEM((4 * PPW,), jnp.float32),
            pltpu.VMEM((6 * PPW,), jnp.float32),
            pltpu.SemaphoreType.DMA,
            pltpu.SemaphoreType.DMA,
            pltpu.SemaphoreType.DMA,
            pltpu.SemaphoreType.DMA,
        ],
    )(_body)
    corr_tile_flat = jnp.transpose(
        corr_m.reshape(B, P // 8, 8, P // 128, 128), (0, 1, 3, 2, 4)
    ).reshape(B * P * P)
    aux = jnp.concatenate([gt_flow, vis_mask, scale_ref],
                          axis=1).reshape(B * 6 * P)
    out = run(corr_tile_flat, aux)
    return out.reshape(B, 6, H, W)
